# initial kernel scaffold (unmeasured)
import jax
import jax.numpy as jnp
from jax import lax
from jax.experimental import pallas as pl
from jax.experimental.pallas import tpu as pltpu


def kernel(
    x,
):
    def body(*refs):
        pass

    out_shape = jax.ShapeDtypeStruct(..., jnp.float32)
    return pl.pallas_call(body, out_shape=out_shape)(...)


# baseline (device time: 2126735 ns/iter reference)
import jax
import jax.numpy as jnp
from jax import lax
from jax.experimental import pallas as pl
from jax.experimental.pallas import tpu as pltpu

N_X = 2


def kernel(x):
    m_per, n = x.shape

    def body(x_ref, out_ref, local_sem, send_sem, recv_sem):
        my_x = lax.axis_index("x")
        my_y = lax.axis_index("y")
        my_z = lax.axis_index("z")
        partner = (1 - my_x, my_y, my_z)

        barrier_sem = pltpu.get_barrier_semaphore()
        pl.semaphore_signal(
            barrier_sem, inc=1,
            device_id=partner, device_id_type=pl.DeviceIdType.MESH,
        )
        pl.semaphore_wait(barrier_sem, 1)

        local = pltpu.make_async_copy(
            x_ref, out_ref.at[pl.ds(my_x * m_per, m_per), :], local_sem
        )
        local.start()

        rdma = pltpu.make_async_remote_copy(
            src_ref=x_ref,
            dst_ref=out_ref.at[pl.ds(my_x * m_per, m_per), :],
            send_sem=send_sem,
            recv_sem=recv_sem,
            device_id=partner,
            device_id_type=pl.DeviceIdType.MESH,
        )
        rdma.start()

        local.wait()
        rdma.wait()

    return pl.pallas_call(
        body,
        out_shape=jax.ShapeDtypeStruct((N_X * m_per, n), x.dtype),
        in_specs=[pl.BlockSpec(memory_space=pl.ANY)],
        out_specs=pl.BlockSpec(memory_space=pl.ANY),
        scratch_shapes=[
            pltpu.SemaphoreType.DMA,
            pltpu.SemaphoreType.DMA,
            pltpu.SemaphoreType.DMA,
        ],
        compiler_params=pltpu.CompilerParams(collective_id=0),
    )(x)


# device time: 811263 ns/iter; 2.6215x vs baseline; 2.6215x over previous
import jax
import jax.numpy as jnp
from jax import lax
from jax.experimental import pallas as pl
from jax.experimental.pallas import tpu as pltpu

N_X = 2
LOCAL_CHUNKS = 16


def kernel(x):
    m_per, n = x.shape
    rows = m_per // LOCAL_CHUNKS

    def body(x_ref, out_ref, vbuf, in_sems, out_sems, send_sem, recv_sem):
        my_x = lax.axis_index("x")
        my_y = lax.axis_index("y")
        my_z = lax.axis_index("z")
        partner = (1 - my_x, my_y, my_z)

        barrier_sem = pltpu.get_barrier_semaphore()
        pl.semaphore_signal(
            barrier_sem, inc=1,
            device_id=partner, device_id_type=pl.DeviceIdType.MESH,
        )
        pl.semaphore_wait(barrier_sem, 1)

        rdma = pltpu.make_async_remote_copy(
            src_ref=x_ref,
            dst_ref=out_ref.at[pl.ds(my_x * m_per, m_per), :],
            send_sem=send_sem,
            recv_sem=recv_sem,
            device_id=partner,
            device_id_type=pl.DeviceIdType.MESH,
        )
        rdma.start()

        loads = [None] * LOCAL_CHUNKS
        stores = [None] * LOCAL_CHUNKS
        for c in range(2):
            ld = pltpu.make_async_copy(
                x_ref.at[pl.ds(c * rows, rows), :], vbuf.at[c], in_sems.at[c]
            )
            ld.start()
            loads[c] = ld
        for c in range(LOCAL_CHUNKS):
            slot = c % 2
            loads[c].wait()
            st = pltpu.make_async_copy(
                vbuf.at[slot],
                out_ref.at[pl.ds(my_x * m_per + c * rows, rows), :],
                out_sems.at[slot],
            )
            st.start()
            stores[c] = st
            nxt = c + 2
            if nxt < LOCAL_CHUNKS:
                stores[c].wait()
                ld = pltpu.make_async_copy(
                    x_ref.at[pl.ds(nxt * rows, rows), :],
                    vbuf.at[nxt % 2],
                    in_sems.at[nxt % 2],
                )
                ld.start()
                loads[nxt] = ld
        for c in range(LOCAL_CHUNKS - 2, LOCAL_CHUNKS):
            stores[c].wait()

        rdma.wait()

    return pl.pallas_call(
        body,
        out_shape=jax.ShapeDtypeStruct((N_X * m_per, n), x.dtype),
        in_specs=[pl.BlockSpec(memory_space=pl.ANY)],
        out_specs=pl.BlockSpec(memory_space=pl.ANY),
        scratch_shapes=[
            pltpu.VMEM((2, rows, n), x.dtype),
            pltpu.SemaphoreType.DMA((2,)),
            pltpu.SemaphoreType.DMA((2,)),
            pltpu.SemaphoreType.DMA,
            pltpu.SemaphoreType.DMA,
        ],
        compiler_params=pltpu.CompilerParams(collective_id=0),
    )(x)


# device time: 503464 ns/iter; 4.2242x vs baseline; 1.6114x over previous
import jax
import jax.numpy as jnp
from jax import lax
from jax.experimental import pallas as pl
from jax.experimental.pallas import tpu as pltpu

N_X = 2
K = 16
LOCAL_CHUNKS = 16


def kernel(x):
    m_per, n = x.shape
    half = m_per // 2
    rows_c = half // K
    rows_l = m_per // LOCAL_CHUNKS

    def body(x_ref, out_ref, vbuf, in_sems, out_sems,
             x_send, x_recv, y_send, y_recv):
        my_x = lax.axis_index("x")
        my_y = lax.axis_index("y")
        my_z = lax.axis_index("z")
        x_partner = (1 - my_x, my_y, my_z)
        y_partner = (my_x, 1 - my_y, my_z)

        barrier_sem = pltpu.get_barrier_semaphore()
        for nbr in (x_partner, y_partner):
            pl.semaphore_signal(
                barrier_sem, inc=1,
                device_id=nbr, device_id_type=pl.DeviceIdType.MESH,
            )
        pl.semaphore_wait(barrier_sem, 2)

        my_half_in = my_x * m_per + my_y * half
        partner_half = (1 - my_x) * m_per + my_y * half

        x_rdmas = []
        for i in range(K):
            r = pltpu.make_async_remote_copy(
                src_ref=x_ref.at[pl.ds(my_y * half + i * rows_c, rows_c), :],
                dst_ref=out_ref.at[pl.ds(my_half_in + i * rows_c, rows_c), :],
                send_sem=x_send.at[i],
                recv_sem=x_recv.at[i],
                device_id=x_partner,
                device_id_type=pl.DeviceIdType.MESH,
            )
            r.start()
            x_rdmas.append(r)

        loads = [None] * LOCAL_CHUNKS
        stores = [None] * LOCAL_CHUNKS
        for c in range(2):
            ld = pltpu.make_async_copy(
                x_ref.at[pl.ds(c * rows_l, rows_l), :], vbuf.at[c],
                in_sems.at[c])
            ld.start()
            loads[c] = ld
        for c in range(LOCAL_CHUNKS):
            loads[c].wait()
            st = pltpu.make_async_copy(
                vbuf.at[c % 2],
                out_ref.at[pl.ds(my_x * m_per + c * rows_l, rows_l), :],
                out_sems.at[c % 2])
            st.start()
            stores[c] = st
            nxt = c + 2
            if nxt < LOCAL_CHUNKS:
                stores[c].wait()
                ld = pltpu.make_async_copy(
                    x_ref.at[pl.ds(nxt * rows_l, rows_l), :],
                    vbuf.at[nxt % 2], in_sems.at[nxt % 2])
                ld.start()
                loads[nxt] = ld
        for c in range(LOCAL_CHUNKS - 2, LOCAL_CHUNKS):
            stores[c].wait()

        y_rdmas = []
        for i in range(K):
            x_rdmas[i].wait_recv()
            r = pltpu.make_async_remote_copy(
                src_ref=out_ref.at[pl.ds(partner_half + i * rows_c, rows_c), :],
                dst_ref=out_ref.at[pl.ds(partner_half + i * rows_c, rows_c), :],
                send_sem=y_send.at[i],
                recv_sem=y_recv.at[i],
                device_id=y_partner,
                device_id_type=pl.DeviceIdType.MESH,
            )
            r.start()
            y_rdmas.append(r)

        for i in range(K):
            x_rdmas[i].wait_send()
            y_rdmas[i].wait()

    return pl.pallas_call(
        body,
        out_shape=jax.ShapeDtypeStruct((N_X * m_per, n), x.dtype),
        in_specs=[pl.BlockSpec(memory_space=pl.ANY)],
        out_specs=pl.BlockSpec(memory_space=pl.ANY),
        scratch_shapes=[
            pltpu.VMEM((2, rows_l, n), x.dtype),
            pltpu.SemaphoreType.DMA((2,)),
            pltpu.SemaphoreType.DMA((2,)),
            pltpu.SemaphoreType.DMA((K,)),
            pltpu.SemaphoreType.DMA((K,)),
            pltpu.SemaphoreType.DMA((K,)),
            pltpu.SemaphoreType.DMA((K,)),
        ],
        compiler_params=pltpu.CompilerParams(collective_id=0),
    )(x)
